# baseline (speedup 1.0000x reference)
import functools

import jax
import jax.numpy as jnp
from jax import lax
from jax.experimental import pallas as pl
from jax.experimental.pallas import tpu as pltpu
from jax.experimental.pallas import tpu_sc as plsc

VOCAB_HID = 64
BATCH = 16384
HIST = 50

NUM_CORES = 2
NUM_SUBCORES = 16
NUM_WORKERS = NUM_CORES * NUM_SUBCORES
BLK = BATCH // NUM_WORKERS
BT_PER_W = BLK // 128

_mesh = plsc.VectorSubcoreMesh(core_axis_name="c", subcore_axis_name="s")


@functools.partial(
    pl.kernel,
    out_type=jax.ShapeDtypeStruct((HIST, 8, 128, 8, 128), jnp.float32),
    mesh=_mesh,
    scratch_types=[
        pltpu.VMEM((HIST, BLK), jnp.int32),
        [pltpu.VMEM((BLK, VOCAB_HID), jnp.float32) for _ in range(2)],
        pltpu.VMEM((8, BT_PER_W, 8, 128), jnp.float32),
        [pltpu.SemaphoreType.DMA for _ in range(2)],
    ],
    compiler_params=pltpu.CompilerParams(use_tc_tiling_on_sc=False,
                                         needs_layout_passes=False),
)
def _gather_kernel(idxT_hbm, table_hbm, out5d, idx_v, rows, tbuf, gsem):
    wid = lax.axis_index("s") * NUM_CORES + lax.axis_index("c")
    b0 = wid * BLK

    pltpu.sync_copy(idxT_hbm.at[:, pl.ds(b0, BLK)], idx_v)

    pltpu.async_copy(table_hbm.at[idx_v.at[0]], rows[0], gsem[0])

    lane = lax.iota(jnp.int32, 16)

    def transpose_block(rows_ref):
        def tr(m, cc):
            ct = m // BT_PER_W
            btl = m % BT_PER_W
            for cs in range(8):
                col = jnp.full((16,), ct * 8 + cs, jnp.int32)
                for k in range(8):
                    rvec = btl * 128 + k * 16 + lane
                    val = plsc.load_gather(rows_ref, [rvec, col])
                    tbuf[ct, btl, cs, pl.ds(k * 16, 16)] = val
            return cc

        lax.fori_loop(0, 8 * BT_PER_W, tr, 0)

    def outer(hh, c):
        for b in range(2):
            h = hh * 2 + b
            pltpu.make_async_copy(table_hbm.at[idx_v.at[h]], rows[b],
                                  gsem[b]).wait()
            hn = h + 1

            @pl.when(hn < HIST)
            def _():
                pltpu.async_copy(table_hbm.at[idx_v.at[hn]], rows[1 - b],
                                 gsem[1 - b])

            transpose_block(rows[b])
            pltpu.sync_copy(tbuf, out5d.at[h, :, pl.ds(wid * BT_PER_W,
                                                       BT_PER_W)])
        return c

    lax.fori_loop(0, HIST // 2, outer, 0)


def kernel(inputs, weight):
    out5d = _gather_kernel(inputs.T, weight)
    return out5d.transpose(2, 4, 0, 1, 3).reshape(BATCH, HIST, VOCAB_HID)


# speedup vs baseline: 1.4967x; 1.4967x over previous
import functools

import jax
import jax.numpy as jnp
from jax import lax
from jax.experimental import pallas as pl
from jax.experimental.pallas import tpu as pltpu
from jax.experimental.pallas import tpu_sc as plsc

HIDDEN = 64
BATCH = 16384
HIST = 50
TOTAL = BATCH * HIST

NUM_CORES = 2
NUM_SUBCORES = 16
NUM_WORKERS = NUM_CORES * NUM_SUBCORES
PER_WORKER = TOTAL // NUM_WORKERS
CHUNK = 256
NCHUNK = PER_WORKER // CHUNK
NBUF = 4
NROUND = NCHUNK // NBUF

_mesh = plsc.VectorSubcoreMesh(core_axis_name="c", subcore_axis_name="s")


@functools.partial(
    pl.kernel,
    out_type=jax.ShapeDtypeStruct((TOTAL, HIDDEN), jnp.float32),
    mesh=_mesh,
    scratch_types=[
        pltpu.VMEM((NCHUNK, CHUNK), jnp.int32),
        [pltpu.VMEM((CHUNK, HIDDEN), jnp.float32) for _ in range(NBUF)],
        [pltpu.SemaphoreType.DMA for _ in range(NBUF)],
        [pltpu.SemaphoreType.DMA for _ in range(NBUF)],
    ],
    compiler_params=pltpu.CompilerParams(use_tc_tiling_on_sc=False),
)
def _gather_kernel(idx_hbm, table_hbm, out_hbm, idx_v, rows, g_sem, w_sem):
    wid = lax.axis_index("s") * NUM_CORES + lax.axis_index("c")
    base = wid * NCHUNK

    def out_slice(i):
        return out_hbm.at[pl.ds((base + i) * CHUNK, CHUNK)]

    pltpu.sync_copy(idx_hbm.at[pl.ds(base, NCHUNK)], idx_v)

    for b in range(NBUF):
        pltpu.async_copy(table_hbm.at[idx_v.at[b]], rows[b], g_sem[b])

    def round_body(r, carry):
        g = r * NBUF
        for b in range(NBUF):
            pltpu.make_async_copy(table_hbm.at[idx_v.at[g + b]], rows[b],
                                  g_sem[b]).wait()
            pltpu.async_copy(rows[b], out_slice(g + b), w_sem[b])
        for b in range(NBUF):
            pltpu.make_async_copy(rows[b], out_slice(g + b), w_sem[b]).wait()
            pltpu.async_copy(table_hbm.at[idx_v.at[g + NBUF + b]], rows[b],
                             g_sem[b])
        return carry

    lax.fori_loop(0, NROUND - 1, round_body, 0)

    g = (NROUND - 1) * NBUF
    for b in range(NBUF):
        pltpu.make_async_copy(table_hbm.at[idx_v.at[g + b]], rows[b],
                              g_sem[b]).wait()
        pltpu.async_copy(rows[b], out_slice(g + b), w_sem[b])
    for b in range(NBUF):
        pltpu.make_async_copy(rows[b], out_slice(g + b), w_sem[b]).wait()


def kernel(inputs, weight):
    idx = inputs.reshape(TOTAL // CHUNK, CHUNK).astype(jnp.int32)
    out = _gather_kernel(idx, weight)
    return out.reshape(inputs.shape + (HIDDEN,))
